# baseline (speedup 1.0000x reference)
import jax
import jax.numpy as jnp
from jax.experimental import pallas as pl

E = 320000
EDGE_BLOCK = 2000


def _conf_body(emb_ref, w1_ref, b1_ref, w2_ref, b2_ref, out_ref):
    emb = emb_ref[...]
    h = jnp.maximum(emb @ w1_ref[...] + b1_ref[...], 0.0)
    out_ref[...] = h @ w2_ref[...] + b2_ref[0, 0]


def _conf_logits(edge_emb, W_ec1, b_ec1, W_ec2, b_ec2):
    grid = (E // EDGE_BLOCK,)
    return pl.pallas_call(
        _conf_body,
        grid=grid,
        in_specs=[
            pl.BlockSpec((EDGE_BLOCK, 256), lambda i: (i, 0)),
            pl.BlockSpec((256, 64), lambda i: (0, 0)),
            pl.BlockSpec((1, 64), lambda i: (0, 0)),
            pl.BlockSpec((64, 1), lambda i: (0, 0)),
            pl.BlockSpec((1, 1), lambda i: (0, 0)),
        ],
        out_specs=pl.BlockSpec((EDGE_BLOCK, 1), lambda i: (i, 0)),
        out_shape=jax.ShapeDtypeStruct((E, 1), jnp.float32),
    )(edge_emb, W_ec1, b_ec1.reshape(1, 64), W_ec2, b_ec2.reshape(1, 1))


def kernel(image_tensor, tweets_tensor, num_prop, category_prop, edge_index, edge_type,
           W_image, b_image, W_tweet, b_tweet, W_num, b_num, W_cat, b_cat,
           W_fusion, b_fusion, edge_type_embed, W_mu, b_mu, W_logvar, b_logvar,
           W_ec1, b_ec1, W_ec2, b_ec2, W_cls, b_cls):
    image_feat = jax.nn.relu(image_tensor @ W_image + b_image)
    tweet_feat = jax.nn.relu(tweets_tensor @ W_tweet + b_tweet)
    num_feat = jax.nn.relu(num_prop @ W_num + b_num)
    category_feat = jax.nn.relu(category_prop @ W_cat + b_cat)
    combined = jnp.concatenate([image_feat, tweet_feat, num_feat, category_feat], axis=-1)
    node_features = jax.nn.relu(combined @ W_fusion + b_fusion)
    src_idx = edge_index[0]
    dst_idx = edge_index[1]
    u_i = node_features[src_idx]
    u_j = node_features[dst_idx]
    edge_feat = jnp.concatenate([u_i, u_j], axis=-1) + edge_type_embed[edge_type]
    edge_emb = jax.nn.relu(edge_feat)
    conf_logits = _conf_logits(edge_emb, W_ec1, b_ec1, W_ec2, b_ec2)[:, 0]
    p_ij = jax.nn.sigmoid(conf_logits)
    rk = jax.random.key(42)
    _, k_bern = jax.random.split(rk)
    u = jax.random.uniform(k_bern, p_ij.shape, dtype=p_ij.dtype)
    s_ij = (u < p_ij).astype(node_features.dtype)
    sum_feat = jax.ops.segment_sum(u_j * s_ij[:, None], src_idx, num_segments=10000)
    count = jax.ops.segment_sum(s_ij, src_idx, num_segments=10000)
    mean_feat = sum_feat / jnp.maximum(count, 1.0)[:, None]
    aggregated = jnp.where(count[:, None] > 0, mean_feat, node_features)
    logits = aggregated @ W_cls + b_cls
    return logits


# speedup vs baseline: 3.3326x; 3.3326x over previous
import functools

import jax
import jax.numpy as jnp
from jax import lax
from jax.experimental import pallas as pl
from jax.experimental.pallas import tpu as pltpu
from jax.experimental.pallas import tpu_sc as plsc

N = 10000
E = 320000
H = 128
EP = 323584
NCHUNK = EP // 128
NW = 32
CPW = NCHUNK // NW
DUMMY = N
NACC = 10112
NB = 1000


def _k1_body(img_ref, tw_ref, nu_ref, ca_ref, wi_ref, bi_ref, wt_ref, bt_ref,
             wn_ref, bn_ref, wc_ref, bc_ref, wf_ref, bf_ref, nf_ref):
    f_im = jnp.maximum(jnp.dot(img_ref[...], wi_ref[...]) + bi_ref[...], 0.0)
    f_tw = jnp.maximum(jnp.dot(tw_ref[...], wt_ref[...]) + bt_ref[...], 0.0)
    f_nu = jnp.maximum(jnp.dot(nu_ref[...], wn_ref[...]) + bn_ref[...], 0.0)
    f_ca = jnp.maximum(jnp.dot(ca_ref[...], wc_ref[...]) + bc_ref[...], 0.0)
    comb = jnp.concatenate([f_im, f_tw, f_nu, f_ca], axis=1)
    nf_ref[...] = jnp.maximum(jnp.dot(comb, wf_ref[...]) + bf_ref[...], 0.0)


def _k1(image, tweets, nu, ca, W_image, b_image, W_tweet, b_tweet,
        W_num, b_num, W_cat, b_cat, W_fusion, b_fusion):
    grid = (N // NB,)
    full = lambda i: (0, 0)
    return pl.pallas_call(
        _k1_body,
        grid=grid,
        in_specs=[
            pl.BlockSpec((NB, 128), lambda i: (i, 0)),
            pl.BlockSpec((NB, 768), lambda i: (i, 0)),
            pl.BlockSpec((NB, 5), lambda i: (i, 0)),
            pl.BlockSpec((NB, 3), lambda i: (i, 0)),
            pl.BlockSpec((128, 128), full),
            pl.BlockSpec((1, 128), full),
            pl.BlockSpec((768, 128), full),
            pl.BlockSpec((1, 128), full),
            pl.BlockSpec((5, 128), full),
            pl.BlockSpec((1, 128), full),
            pl.BlockSpec((3, 128), full),
            pl.BlockSpec((1, 128), full),
            pl.BlockSpec((512, 128), full),
            pl.BlockSpec((1, 128), full),
        ],
        out_specs=pl.BlockSpec((NB, 128), lambda i: (i, 0)),
        out_shape=jax.ShapeDtypeStruct((N, 128), jnp.float32),
    )(image, tweets, nu, ca, W_image, b_image, W_tweet, b_tweet,
      W_num, b_num, W_cat, b_cat, W_fusion, b_fusion)


def _k2(nf, ia, ib):
    mesh = plsc.VectorSubcoreMesh(core_axis_name="c", subcore_axis_name="s")
    ia2 = ia.reshape(1, EP)
    ib2 = ib.reshape(1, EP)

    @functools.partial(
        pl.kernel,
        out_type=[jax.ShapeDtypeStruct((EP, 128), jnp.float32),
                  jax.ShapeDtypeStruct((EP, 128), jnp.float32)],
        mesh=mesh)
    def k2(t_hbm, ia_hbm, ib_hbm, oa_hbm, ob_hbm):
        def body(ia_v, ib_v, oa_v, ob_v):
            pltpu.sync_copy(t_hbm.at[ia_v.at[0]], oa_v)
            pltpu.sync_copy(t_hbm.at[ib_v.at[0]], ob_v)

        pltpu.emit_pipeline(
            body,
            grid=(NCHUNK,),
            in_specs=[pl.BlockSpec((1, 128), lambda i: (0, i)),
                      pl.BlockSpec((1, 128), lambda i: (0, i))],
            out_specs=[pl.BlockSpec((128, 128), lambda i: (i, 0)),
                       pl.BlockSpec((128, 128), lambda i: (i, 0))],
            core_axis_name=("c", "s"),
            dimension_semantics=(pltpu.PARALLEL,),
        )(ia_hbm, ib_hbm, oa_hbm, ob_hbm)

    return k2(nf, ia2, ib2)


EB = 4096


def _k3_body(g1_ref, g2_ref, et_ref, emb_ref, w1_ref, b1_ref, w2_ref, b2_ref, conf_ref):
    cat = jnp.concatenate([g1_ref[...], g2_ref[...]], axis=1)
    et = et_ref[...]
    emb = jnp.where(et == 0, emb_ref[0, :][None, :],
                    jnp.where(et == 1, emb_ref[1, :][None, :], emb_ref[2, :][None, :]))
    edge_emb = jnp.maximum(cat + emb, 0.0)
    h = jnp.maximum(jnp.dot(edge_emb, w1_ref[...]) + b1_ref[...], 0.0)
    conf_ref[...] = jnp.dot(h, w2_ref[...]) + b2_ref[0, 0]


def _k3(g1, g2, et, et_embed, W_ec1, b_ec1, W_ec2, b_ec2):
    grid = (EP // EB,)
    return pl.pallas_call(
        _k3_body,
        grid=grid,
        in_specs=[
            pl.BlockSpec((EB, 128), lambda i: (i, 0)),
            pl.BlockSpec((EB, 128), lambda i: (i, 0)),
            pl.BlockSpec((EB, 1), lambda i: (i, 0)),
            pl.BlockSpec((3, 256), lambda i: (0, 0)),
            pl.BlockSpec((256, 64), lambda i: (0, 0)),
            pl.BlockSpec((1, 64), lambda i: (0, 0)),
            pl.BlockSpec((64, 1), lambda i: (0, 0)),
            pl.BlockSpec((1, 1), lambda i: (0, 0)),
        ],
        out_specs=pl.BlockSpec((EB, 1), lambda i: (i, 0)),
        out_shape=jax.ShapeDtypeStruct((EP, 1), jnp.float32),
    )(g1, g2, et, et_embed, W_ec1, b_ec1.reshape(1, 64), W_ec2, b_ec2.reshape(1, 1))


def _k4(uj, mi):
    mesh = plsc.VectorSubcoreMesh(core_axis_name="c", subcore_axis_name="s")

    @functools.partial(
        pl.kernel,
        out_type=[jax.ShapeDtypeStruct((2, NACC, 128), jnp.float32),
                  jax.ShapeDtypeStruct((2, NACC, 128), jnp.float32)],
        mesh=mesh,
        scratch_types=[
            pltpu.VMEM((128, 128), jnp.float32),
            pltpu.VMEM((128,), jnp.int32),
            pltpu.VMEM((128, 128), jnp.float32),
            pltpu.VMEM((128, 128), jnp.float32),
            pltpu.VMEM_SHARED((NACC, 128), jnp.float32),
        ])
    def k4(uj_hbm, mi_hbm, outf_hbm, outc_hbm, rows_v, mi_v, zero_v, ones_v, acc):
        cid = lax.axis_index("c")
        sid = lax.axis_index("s")
        wid = sid * 2 + cid

        @pl.loop(0, 128)
        def _(r):
            @pl.loop(0, 128, step=16)
            def _(c0):
                zero_v[r, pl.ds(c0, 16)] = jnp.zeros((16,), jnp.float32)
                ones_v[r, pl.ds(c0, 16)] = jnp.ones((16,), jnp.float32)

        def zero_acc():
            @pl.loop(0, 5)
            def _(j):
                k = sid + j * 16

                @pl.when(k < CPW)
                def _():
                    pltpu.sync_copy(zero_v, acc.at[pl.ds(k * 128, 128)])

        def writeback(out_hbm):
            @pl.loop(0, 5)
            def _(j):
                k = sid + j * 16

                @pl.when(k < CPW)
                def _():
                    pltpu.sync_copy(acc.at[pl.ds(k * 128, 128)],
                                    out_hbm.at[cid].at[pl.ds(k * 128, 128)])

        zero_acc()
        plsc.subcore_barrier()

        @pl.loop(0, CPW)
        def _(j):
            c = wid * CPW + j
            pltpu.sync_copy(mi_hbm.at[pl.ds(c * 128, 128)], mi_v)
            pltpu.sync_copy(uj_hbm.at[pl.ds(c * 128, 128)], rows_v)
            pltpu.sync_copy(rows_v, acc.at[mi_v], add=True)

        plsc.subcore_barrier()
        writeback(outf_hbm)
        plsc.subcore_barrier()

        zero_acc()
        plsc.subcore_barrier()

        @pl.loop(0, CPW)
        def _(j):
            c = wid * CPW + j
            pltpu.sync_copy(mi_hbm.at[pl.ds(c * 128, 128)], mi_v)
            pltpu.sync_copy(ones_v, acc.at[mi_v], add=True)

        plsc.subcore_barrier()
        writeback(outc_hbm)

    return k4(uj, mi)


def _k5_body(pf0_ref, pf1_ref, pc0_ref, pc1_ref, nf_ref, wcls_ref, bcls_ref, out_ref):
    sf = pf0_ref[...] + pf1_ref[...]
    cnt = pc0_ref[:, 0:1] + pc1_ref[:, 0:1]
    mean = sf / jnp.maximum(cnt, 1.0)
    agg = jnp.where(cnt > 0, mean, nf_ref[...])
    out_ref[...] = jnp.dot(agg, wcls_ref[...]) + bcls_ref[...]


def _k5(pf0, pf1, pc0, pc1, nf, W_cls, b_cls):
    grid = (N // NB,)
    return pl.pallas_call(
        _k5_body,
        grid=grid,
        in_specs=[
            pl.BlockSpec((NB, 128), lambda i: (i, 0)),
            pl.BlockSpec((NB, 128), lambda i: (i, 0)),
            pl.BlockSpec((NB, 128), lambda i: (i, 0)),
            pl.BlockSpec((NB, 128), lambda i: (i, 0)),
            pl.BlockSpec((NB, 128), lambda i: (i, 0)),
            pl.BlockSpec((128, 2), lambda i: (0, 0)),
            pl.BlockSpec((1, 2), lambda i: (0, 0)),
        ],
        out_specs=pl.BlockSpec((NB, 2), lambda i: (i, 0)),
        out_shape=jax.ShapeDtypeStruct((N, 2), jnp.float32),
    )(pf0, pf1, pc0, pc1, nf, W_cls, b_cls.reshape(1, 2))


def kernel(image_tensor, tweets_tensor, num_prop, category_prop, edge_index, edge_type,
           W_image, b_image, W_tweet, b_tweet, W_num, b_num, W_cat, b_cat,
           W_fusion, b_fusion, edge_type_embed, W_mu, b_mu, W_logvar, b_logvar,
           W_ec1, b_ec1, W_ec2, b_ec2, W_cls, b_cls):
    nf = _k1(image_tensor, tweets_tensor, num_prop, category_prop,
             W_image, b_image.reshape(1, H), W_tweet, b_tweet.reshape(1, H),
             W_num, b_num.reshape(1, H), W_cat, b_cat.reshape(1, H),
             W_fusion, b_fusion.reshape(1, H))

    src = edge_index[0]
    dst = edge_index[1]
    pad = EP - E
    src_p = jnp.pad(src, (0, pad))
    dst_p = jnp.pad(dst, (0, pad))
    et_p = jnp.pad(edge_type, (0, pad))

    g1, g2 = _k2(nf, src_p, dst_p)

    conf = _k3(g1, g2, et_p.reshape(EP, 1), edge_type_embed, W_ec1, b_ec1, W_ec2, b_ec2)

    _, k_bern = jax.random.split(jax.random.key(42))
    u = jax.random.uniform(k_bern, (E,), dtype=jnp.float32)
    u_p = jnp.pad(u, (0, pad), constant_values=2.0)
    p_ij = jax.nn.sigmoid(conf[:, 0])
    s = u_p < p_ij
    mi = jnp.where(s, src_p, DUMMY).astype(jnp.int32)

    pf, pc = _k4(g2, mi)

    return _k5(pf[0], pf[1], pc[0], pc[1], nf, W_cls, b_cls)


# speedup vs baseline: 3.5676x; 1.0705x over previous
import functools

import jax
import jax.numpy as jnp
from jax import lax
from jax.experimental import pallas as pl
from jax.experimental.pallas import tpu as pltpu
from jax.experimental.pallas import tpu_sc as plsc

N = 10000
E = 320000
H = 128
EP = 323584
NCHUNK = EP // 128
NW = 32
CPW = NCHUNK // NW
DUMMY = N
NACC = 10112
NB = 1000


def _k1_body(img_ref, tw_ref, nu_ref, ca_ref, wi_ref, bi_ref, wt_ref, bt_ref,
             wn_ref, bn_ref, wc_ref, bc_ref, wf_ref, bf_ref, nf_ref):
    f_im = jnp.maximum(jnp.dot(img_ref[...], wi_ref[...]) + bi_ref[...], 0.0)
    f_tw = jnp.maximum(jnp.dot(tw_ref[...], wt_ref[...]) + bt_ref[...], 0.0)
    f_nu = jnp.maximum(jnp.dot(nu_ref[...], wn_ref[...]) + bn_ref[...], 0.0)
    f_ca = jnp.maximum(jnp.dot(ca_ref[...], wc_ref[...]) + bc_ref[...], 0.0)
    comb = jnp.concatenate([f_im, f_tw, f_nu, f_ca], axis=1)
    nf_ref[...] = jnp.maximum(jnp.dot(comb, wf_ref[...]) + bf_ref[...], 0.0)


def _k1(image, tweets, nu, ca, W_image, b_image, W_tweet, b_tweet,
        W_num, b_num, W_cat, b_cat, W_fusion, b_fusion):
    grid = (N // NB,)
    full = lambda i: (0, 0)
    return pl.pallas_call(
        _k1_body,
        grid=grid,
        in_specs=[
            pl.BlockSpec((NB, 128), lambda i: (i, 0)),
            pl.BlockSpec((NB, 768), lambda i: (i, 0)),
            pl.BlockSpec((NB, 5), lambda i: (i, 0)),
            pl.BlockSpec((NB, 3), lambda i: (i, 0)),
            pl.BlockSpec((128, 128), full),
            pl.BlockSpec((1, 128), full),
            pl.BlockSpec((768, 128), full),
            pl.BlockSpec((1, 128), full),
            pl.BlockSpec((5, 128), full),
            pl.BlockSpec((1, 128), full),
            pl.BlockSpec((3, 128), full),
            pl.BlockSpec((1, 128), full),
            pl.BlockSpec((512, 128), full),
            pl.BlockSpec((1, 128), full),
        ],
        out_specs=pl.BlockSpec((NB, 128), lambda i: (i, 0)),
        out_shape=jax.ShapeDtypeStruct((N, 128), jnp.float32),
    )(image, tweets, nu, ca, W_image, b_image, W_tweet, b_tweet,
      W_num, b_num, W_cat, b_cat, W_fusion, b_fusion)


def _k2(nf, ia, ib):
    mesh = plsc.VectorSubcoreMesh(core_axis_name="c", subcore_axis_name="s")

    @functools.partial(
        pl.kernel,
        out_type=[jax.ShapeDtypeStruct((EP, 128), jnp.float32),
                  jax.ShapeDtypeStruct((EP, 128), jnp.float32)],
        mesh=mesh,
        scratch_types=[
            pltpu.VMEM((2, 128), jnp.int32),
            pltpu.VMEM((2, 128), jnp.int32),
            pltpu.VMEM((2, 128, 128), jnp.float32),
            pltpu.VMEM((2, 128, 128), jnp.float32),
            pltpu.SemaphoreType.DMA,
            pltpu.SemaphoreType.DMA,
            pltpu.SemaphoreType.DMA,
            pltpu.SemaphoreType.DMA,
            pltpu.SemaphoreType.DMA,
            pltpu.SemaphoreType.DMA,
        ])
    def k2(nf_hbm, ia_hbm, ib_hbm, oa_hbm, ob_hbm,
           iav, ibv, rav, rbv, si0, si1, sg0, sg1, sw0, sw1):
        wid = lax.axis_index("s") * 2 + lax.axis_index("c")
        base = wid * CPW
        si = (si0, si1)
        sg = (sg0, sg1)
        sw = (sw0, sw1)

        def idx_issue(c, buf):
            pltpu.async_copy(ia_hbm.at[pl.ds(c * 128, 128)], iav.at[buf], si[buf])
            pltpu.async_copy(ib_hbm.at[pl.ds(c * 128, 128)], ibv.at[buf], si[buf])

        def idx_wait(c, buf):
            pltpu.make_async_copy(ia_hbm.at[pl.ds(c * 128, 128)], iav.at[buf], si[buf]).wait()
            pltpu.make_async_copy(ib_hbm.at[pl.ds(c * 128, 128)], ibv.at[buf], si[buf]).wait()

        def g_issue(buf):
            pltpu.async_copy(nf_hbm.at[iav.at[buf]], rav.at[buf], sg[buf])
            pltpu.async_copy(nf_hbm.at[ibv.at[buf]], rbv.at[buf], sg[buf])

        def g_wait(buf):
            pltpu.make_async_copy(nf_hbm.at[iav.at[buf]], rav.at[buf], sg[buf]).wait()
            pltpu.make_async_copy(nf_hbm.at[ibv.at[buf]], rbv.at[buf], sg[buf]).wait()

        def w_issue(c, buf):
            pltpu.async_copy(rav.at[buf], oa_hbm.at[pl.ds(c * 128, 128)], sw[buf])
            pltpu.async_copy(rbv.at[buf], ob_hbm.at[pl.ds(c * 128, 128)], sw[buf])

        def w_wait(c, buf):
            pltpu.make_async_copy(rav.at[buf], oa_hbm.at[pl.ds(c * 128, 128)], sw[buf]).wait()
            pltpu.make_async_copy(rbv.at[buf], ob_hbm.at[pl.ds(c * 128, 128)], sw[buf]).wait()

        idx_issue(base, 0)

        @pl.loop(0, CPW)
        def _(j):
            c = base + j
            for buf in (0, 1):
                obuf = 1 - buf

                @pl.when(lax.rem(j, 2) == buf)
                def _():
                    @pl.when(j >= 2)
                    def _():
                        w_wait(c - 2, buf)

                    idx_wait(c, buf)
                    g_issue(buf)

                    @pl.when(j >= 1)
                    def _():
                        g_wait(obuf)
                        w_issue(c - 1, obuf)

                    @pl.when(j + 1 < CPW)
                    def _():
                        idx_issue(c + 1, obuf)

        lastbuf = (CPW - 1) % 2
        g_wait(lastbuf)
        w_issue(base + CPW - 1, lastbuf)
        w_wait(base + CPW - 1, lastbuf)
        w_wait(base + CPW - 2, 1 - lastbuf)

    return k2(nf, ia, ib)


EB = 4096


def _k3_body(g1_ref, g2_ref, et_ref, emb_ref, w1_ref, b1_ref, w2_ref, b2_ref, conf_ref):
    cat = jnp.concatenate([g1_ref[...], g2_ref[...]], axis=1)
    et = et_ref[...]
    emb = jnp.where(et == 0, emb_ref[0, :][None, :],
                    jnp.where(et == 1, emb_ref[1, :][None, :], emb_ref[2, :][None, :]))
    edge_emb = jnp.maximum(cat + emb, 0.0)
    h = jnp.maximum(jnp.dot(edge_emb, w1_ref[...]) + b1_ref[...], 0.0)
    conf_ref[...] = jnp.dot(h, w2_ref[...]) + b2_ref[0, 0]


def _k3(g1, g2, et, et_embed, W_ec1, b_ec1, W_ec2, b_ec2):
    grid = (EP // EB,)
    return pl.pallas_call(
        _k3_body,
        grid=grid,
        in_specs=[
            pl.BlockSpec((EB, 128), lambda i: (i, 0)),
            pl.BlockSpec((EB, 128), lambda i: (i, 0)),
            pl.BlockSpec((EB, 1), lambda i: (i, 0)),
            pl.BlockSpec((3, 256), lambda i: (0, 0)),
            pl.BlockSpec((256, 64), lambda i: (0, 0)),
            pl.BlockSpec((1, 64), lambda i: (0, 0)),
            pl.BlockSpec((64, 1), lambda i: (0, 0)),
            pl.BlockSpec((1, 1), lambda i: (0, 0)),
        ],
        out_specs=pl.BlockSpec((EB, 1), lambda i: (i, 0)),
        out_shape=jax.ShapeDtypeStruct((EP, 1), jnp.float32),
    )(g1, g2, et, et_embed, W_ec1, b_ec1.reshape(1, 64), W_ec2, b_ec2.reshape(1, 1))


def _k4(uj, mi):
    mesh = plsc.VectorSubcoreMesh(core_axis_name="c", subcore_axis_name="s")

    @functools.partial(
        pl.kernel,
        out_type=[jax.ShapeDtypeStruct((2, NACC, 128), jnp.float32),
                  jax.ShapeDtypeStruct((2, NACC, 128), jnp.float32)],
        mesh=mesh,
        scratch_types=[
            pltpu.VMEM((2, 128, 128), jnp.float32),
            pltpu.VMEM((2, 128), jnp.int32),
            pltpu.VMEM_SHARED((NACC, 128), jnp.float32),
            pltpu.SemaphoreType.DMA,
            pltpu.SemaphoreType.DMA,
        ])
    def k4(uj_hbm, mi_hbm, outf_hbm, outc_hbm, rows_v, mi_v, acc,
           sin0, sin1):
        cid = lax.axis_index("c")
        sid = lax.axis_index("s")
        wid = sid * 2 + cid
        base = wid * CPW
        sin = (sin0, sin1)

        def fill0(val):
            @pl.loop(0, 128)
            def _(r):
                @pl.loop(0, 128, step=16)
                def _(c0):
                    rows_v[0, r, pl.ds(c0, 16)] = jnp.full((16,), val, jnp.float32)

        def zero_acc():
            @pl.loop(0, 5)
            def _(j):
                k = sid + j * 16

                @pl.when(k < CPW)
                def _():
                    pltpu.sync_copy(rows_v.at[0], acc.at[pl.ds(k * 128, 128)])

        def writeback(out_hbm):
            @pl.loop(0, 5)
            def _(j):
                k = sid + j * 16

                @pl.when(k < CPW)
                def _():
                    pltpu.sync_copy(acc.at[pl.ds(k * 128, 128)],
                                    out_hbm.at[cid].at[pl.ds(k * 128, 128)])

        def a_issue(c, buf):
            pltpu.async_copy(mi_hbm.at[pl.ds(c * 128, 128)], mi_v.at[buf], sin[buf])
            pltpu.async_copy(uj_hbm.at[pl.ds(c * 128, 128)], rows_v.at[buf], sin[buf])

        def a_wait(c, buf):
            pltpu.make_async_copy(mi_hbm.at[pl.ds(c * 128, 128)], mi_v.at[buf], sin[buf]).wait()
            pltpu.make_async_copy(uj_hbm.at[pl.ds(c * 128, 128)], rows_v.at[buf], sin[buf]).wait()

        def b_issue(c, buf):
            pltpu.async_copy(mi_hbm.at[pl.ds(c * 128, 128)], mi_v.at[buf], sin[buf])

        def b_wait(c, buf):
            pltpu.make_async_copy(mi_hbm.at[pl.ds(c * 128, 128)], mi_v.at[buf], sin[buf]).wait()

        fill0(0.0)
        zero_acc()
        plsc.subcore_barrier()
        a_issue(base, 0)

        @pl.loop(0, CPW)
        def _(j):
            c = base + j
            for buf in (0, 1):
                obuf = 1 - buf

                @pl.when(lax.rem(j, 2) == buf)
                def _():
                    a_wait(c, buf)

                    @pl.when(j + 1 < CPW)
                    def _():
                        a_issue(c + 1, obuf)

                    pltpu.sync_copy(rows_v.at[buf], acc.at[mi_v.at[buf]], add=True)

        plsc.subcore_barrier()
        writeback(outf_hbm)
        plsc.subcore_barrier()

        fill0(0.0)
        zero_acc()
        fill0(1.0)
        plsc.subcore_barrier()
        b_issue(base, 0)

        @pl.loop(0, CPW)
        def _(j):
            c = base + j
            for buf in (0, 1):
                obuf = 1 - buf

                @pl.when(lax.rem(j, 2) == buf)
                def _():
                    b_wait(c, buf)

                    @pl.when(j + 1 < CPW)
                    def _():
                        b_issue(c + 1, obuf)

                    pltpu.sync_copy(rows_v.at[0], acc.at[mi_v.at[buf]], add=True)

        plsc.subcore_barrier()
        writeback(outc_hbm)

    return k4(uj, mi)


def _k5_body(pf0_ref, pf1_ref, pc0_ref, pc1_ref, nf_ref, wcls_ref, bcls_ref, out_ref):
    sf = pf0_ref[...] + pf1_ref[...]
    cnt = pc0_ref[:, 0:1] + pc1_ref[:, 0:1]
    mean = sf / jnp.maximum(cnt, 1.0)
    agg = jnp.where(cnt > 0, mean, nf_ref[...])
    out_ref[...] = jnp.dot(agg, wcls_ref[...]) + bcls_ref[...]


def _k5(pf0, pf1, pc0, pc1, nf, W_cls, b_cls):
    grid = (N // NB,)
    return pl.pallas_call(
        _k5_body,
        grid=grid,
        in_specs=[
            pl.BlockSpec((NB, 128), lambda i: (i, 0)),
            pl.BlockSpec((NB, 128), lambda i: (i, 0)),
            pl.BlockSpec((NB, 128), lambda i: (i, 0)),
            pl.BlockSpec((NB, 128), lambda i: (i, 0)),
            pl.BlockSpec((NB, 128), lambda i: (i, 0)),
            pl.BlockSpec((128, 2), lambda i: (0, 0)),
            pl.BlockSpec((1, 2), lambda i: (0, 0)),
        ],
        out_specs=pl.BlockSpec((NB, 2), lambda i: (i, 0)),
        out_shape=jax.ShapeDtypeStruct((N, 2), jnp.float32),
    )(pf0, pf1, pc0, pc1, nf, W_cls, b_cls.reshape(1, 2))


def kernel(image_tensor, tweets_tensor, num_prop, category_prop, edge_index, edge_type,
           W_image, b_image, W_tweet, b_tweet, W_num, b_num, W_cat, b_cat,
           W_fusion, b_fusion, edge_type_embed, W_mu, b_mu, W_logvar, b_logvar,
           W_ec1, b_ec1, W_ec2, b_ec2, W_cls, b_cls):
    nf = _k1(image_tensor, tweets_tensor, num_prop, category_prop,
             W_image, b_image.reshape(1, H), W_tweet, b_tweet.reshape(1, H),
             W_num, b_num.reshape(1, H), W_cat, b_cat.reshape(1, H),
             W_fusion, b_fusion.reshape(1, H))

    src = edge_index[0]
    dst = edge_index[1]
    pad = EP - E
    src_p = jnp.pad(src, (0, pad))
    dst_p = jnp.pad(dst, (0, pad))
    et_p = jnp.pad(edge_type, (0, pad))

    g1, g2 = _k2(nf, src_p, dst_p)

    conf = _k3(g1, g2, et_p.reshape(EP, 1), edge_type_embed, W_ec1, b_ec1, W_ec2, b_ec2)

    _, k_bern = jax.random.split(jax.random.key(42))
    u = jax.random.uniform(k_bern, (E,), dtype=jnp.float32)
    u_p = jnp.pad(u, (0, pad), constant_values=2.0)
    p_ij = jax.nn.sigmoid(conf[:, 0])
    s = u_p < p_ij
    mi = jnp.where(s, src_p, DUMMY).astype(jnp.int32)

    pf, pc = _k4(g2, mi)

    return _k5(pf[0], pf[1], pc[0], pc[1], nf, W_cls, b_cls)


# speedup vs baseline: 4.5495x; 1.2752x over previous
import dataclasses
import functools

import jax
import jax.numpy as jnp
from jax import lax
from jax.experimental import pallas as pl
from jax.experimental.pallas import tpu as pltpu
from jax.experimental.pallas import tpu_sc as plsc

N = 10000
E = 320000
H = 128
EP = 323584
NCHUNK = EP // 128
NW = 32
CPW = NCHUNK // NW
DUMMY = N
NACC = 10112
NB = 1000


def _k1_body(img_ref, tw_ref, nu_ref, ca_ref, wi_ref, bi_ref, wt_ref, bt_ref,
             wn_ref, bn_ref, wc_ref, bc_ref, wf_ref, bf_ref, nf_ref):
    f_im = jnp.maximum(jnp.dot(img_ref[...], wi_ref[...]) + bi_ref[...], 0.0)
    f_tw = jnp.maximum(jnp.dot(tw_ref[...], wt_ref[...]) + bt_ref[...], 0.0)
    f_nu = jnp.maximum(jnp.dot(nu_ref[...], wn_ref[...]) + bn_ref[...], 0.0)
    f_ca = jnp.maximum(jnp.dot(ca_ref[...], wc_ref[...]) + bc_ref[...], 0.0)
    comb = jnp.concatenate([f_im, f_tw, f_nu, f_ca], axis=1)
    nf_ref[...] = jnp.maximum(jnp.dot(comb, wf_ref[...]) + bf_ref[...], 0.0)


def _k1(image, tweets, nu, ca, W_image, b_image, W_tweet, b_tweet,
        W_num, b_num, W_cat, b_cat, W_fusion, b_fusion):
    grid = (N // NB,)
    full = lambda i: (0, 0)
    return pl.pallas_call(
        _k1_body,
        grid=grid,
        in_specs=[
            pl.BlockSpec((NB, 128), lambda i: (i, 0)),
            pl.BlockSpec((NB, 768), lambda i: (i, 0)),
            pl.BlockSpec((NB, 5), lambda i: (i, 0)),
            pl.BlockSpec((NB, 3), lambda i: (i, 0)),
            pl.BlockSpec((128, 128), full),
            pl.BlockSpec((1, 128), full),
            pl.BlockSpec((768, 128), full),
            pl.BlockSpec((1, 128), full),
            pl.BlockSpec((5, 128), full),
            pl.BlockSpec((1, 128), full),
            pl.BlockSpec((3, 128), full),
            pl.BlockSpec((1, 128), full),
            pl.BlockSpec((512, 128), full),
            pl.BlockSpec((1, 128), full),
        ],
        out_specs=pl.BlockSpec((NB, 128), lambda i: (i, 0)),
        out_shape=jax.ShapeDtypeStruct((N, 128), jnp.float32),
    )(image, tweets, nu, ca, W_image, b_image, W_tweet, b_tweet,
      W_num, b_num, W_cat, b_cat, W_fusion, b_fusion)


def _k2(nf, ia, ib):
    mesh = plsc.VectorSubcoreMesh(core_axis_name="c", subcore_axis_name="s")

    @functools.partial(
        pl.kernel,
        out_type=[jax.ShapeDtypeStruct((EP, 128), jnp.float32),
                  jax.ShapeDtypeStruct((EP, 128), jnp.float32)],
        mesh=mesh,
        scratch_types=[
            pltpu.VMEM((2, 128), jnp.int32),
            pltpu.VMEM((2, 128), jnp.int32),
            pltpu.VMEM((2, 128, 128), jnp.float32),
            pltpu.VMEM((2, 128, 128), jnp.float32),
            pltpu.SemaphoreType.DMA,
            pltpu.SemaphoreType.DMA,
            pltpu.SemaphoreType.DMA,
            pltpu.SemaphoreType.DMA,
            pltpu.SemaphoreType.DMA,
            pltpu.SemaphoreType.DMA,
        ])
    def k2(nf_hbm, ia_hbm, ib_hbm, oa_hbm, ob_hbm,
           iav, ibv, rav, rbv, si0, si1, sg0, sg1, sw0, sw1):
        wid = lax.axis_index("s") * 2 + lax.axis_index("c")
        base = wid * CPW
        si = (si0, si1)
        sg = (sg0, sg1)
        sw = (sw0, sw1)

        def idx_issue(c, buf):
            pltpu.async_copy(ia_hbm.at[pl.ds(c * 128, 128)], iav.at[buf], si[buf])
            pltpu.async_copy(ib_hbm.at[pl.ds(c * 128, 128)], ibv.at[buf], si[buf])

        def idx_wait(c, buf):
            pltpu.make_async_copy(ia_hbm.at[pl.ds(c * 128, 128)], iav.at[buf], si[buf]).wait()
            pltpu.make_async_copy(ib_hbm.at[pl.ds(c * 128, 128)], ibv.at[buf], si[buf]).wait()

        def g_issue(buf):
            pltpu.async_copy(nf_hbm.at[iav.at[buf]], rav.at[buf], sg[buf])
            pltpu.async_copy(nf_hbm.at[ibv.at[buf]], rbv.at[buf], sg[buf])

        def g_wait(buf):
            pltpu.make_async_copy(nf_hbm.at[iav.at[buf]], rav.at[buf], sg[buf]).wait()
            pltpu.make_async_copy(nf_hbm.at[ibv.at[buf]], rbv.at[buf], sg[buf]).wait()

        def w_issue(c, buf):
            pltpu.async_copy(rav.at[buf], oa_hbm.at[pl.ds(c * 128, 128)], sw[buf])
            pltpu.async_copy(rbv.at[buf], ob_hbm.at[pl.ds(c * 128, 128)], sw[buf])

        def w_wait(c, buf):
            pltpu.make_async_copy(rav.at[buf], oa_hbm.at[pl.ds(c * 128, 128)], sw[buf]).wait()
            pltpu.make_async_copy(rbv.at[buf], ob_hbm.at[pl.ds(c * 128, 128)], sw[buf]).wait()

        idx_issue(base, 0)

        @pl.loop(0, CPW)
        def _(j):
            c = base + j
            for buf in (0, 1):
                obuf = 1 - buf

                @pl.when(lax.rem(j, 2) == buf)
                def _():
                    @pl.when(j >= 2)
                    def _():
                        w_wait(c - 2, buf)

                    idx_wait(c, buf)
                    g_issue(buf)

                    @pl.when(j >= 1)
                    def _():
                        g_wait(obuf)
                        w_issue(c - 1, obuf)

                    @pl.when(j + 1 < CPW)
                    def _():
                        idx_issue(c + 1, obuf)

        lastbuf = (CPW - 1) % 2
        g_wait(lastbuf)
        w_issue(base + CPW - 1, lastbuf)
        w_wait(base + CPW - 1, lastbuf)
        w_wait(base + CPW - 2, 1 - lastbuf)

    return k2(nf, ia, ib)


EB = 4096


def _k3_body(g1_ref, g2_ref, et_ref, emb_ref, w1_ref, b1_ref, w2_ref, b2_ref, conf_ref):
    cat = jnp.concatenate([g1_ref[...], g2_ref[...]], axis=1)
    et = et_ref[...]
    emb = jnp.where(et == 0, emb_ref[0, :][None, :],
                    jnp.where(et == 1, emb_ref[1, :][None, :], emb_ref[2, :][None, :]))
    edge_emb = jnp.maximum(cat + emb, 0.0)
    h = jnp.maximum(jnp.dot(edge_emb, w1_ref[...]) + b1_ref[...], 0.0)
    conf_ref[...] = jnp.dot(h, w2_ref[...]) + b2_ref[0, 0]


def _k3(g1, g2, et, et_embed, W_ec1, b_ec1, W_ec2, b_ec2):
    grid = (EP // EB,)
    return pl.pallas_call(
        _k3_body,
        grid=grid,
        in_specs=[
            pl.BlockSpec((EB, 128), lambda i: (i, 0)),
            pl.BlockSpec((EB, 128), lambda i: (i, 0)),
            pl.BlockSpec((EB, 1), lambda i: (i, 0)),
            pl.BlockSpec((3, 256), lambda i: (0, 0)),
            pl.BlockSpec((256, 64), lambda i: (0, 0)),
            pl.BlockSpec((1, 64), lambda i: (0, 0)),
            pl.BlockSpec((64, 1), lambda i: (0, 0)),
            pl.BlockSpec((1, 1), lambda i: (0, 0)),
        ],
        out_specs=pl.BlockSpec((EB, 1), lambda i: (i, 0)),
        out_shape=jax.ShapeDtypeStruct((EP, 1), jnp.float32),
    )(g1, g2, et, et_embed, W_ec1, b_ec1.reshape(1, 64), W_ec2, b_ec2.reshape(1, 1))


def _k4(uj, mi):
    mesh = plsc.VectorSubcoreMesh(core_axis_name="c", subcore_axis_name="s")
    cp = pltpu.CompilerParams()
    if "needs_layout_passes" in pltpu.CompilerParams.__dataclass_fields__:
        cp = dataclasses.replace(cp, needs_layout_passes=False)

    @functools.partial(
        pl.kernel,
        out_type=[jax.ShapeDtypeStruct((2, NACC, 128), jnp.float32),
                  jax.ShapeDtypeStruct((NW, 80, 128), jnp.float32)],
        mesh=mesh,
        compiler_params=cp,
        scratch_types=[
            pltpu.VMEM((2, 128, 128), jnp.float32),
            pltpu.VMEM((2, 128), jnp.int32),
            pltpu.VMEM((80, 128), jnp.float32),
            pltpu.VMEM_SHARED((NACC, 128), jnp.float32),
            pltpu.SemaphoreType.DMA,
            pltpu.SemaphoreType.DMA,
        ])
    def k4(uj_hbm, mi_hbm, outf_hbm, outc_hbm, rows_v, mi_v, cnt_v, acc,
           sin0, sin1):
        cid = lax.axis_index("c")
        sid = lax.axis_index("s")
        wid = sid * 2 + cid
        base = wid * CPW
        sin = (sin0, sin1)

        def fill0(val):
            @pl.loop(0, 128)
            def _(r):
                @pl.loop(0, 128, step=16)
                def _(c0):
                    rows_v[0, r, pl.ds(c0, 16)] = jnp.full((16,), val, jnp.float32)

        @pl.loop(0, 80)
        def _(r):
            @pl.loop(0, 128, step=16)
            def _(c0):
                cnt_v[r, pl.ds(c0, 16)] = jnp.zeros((16,), jnp.float32)

        def zero_acc():
            @pl.loop(0, 5)
            def _(j):
                k = sid + j * 16

                @pl.when(k < CPW)
                def _():
                    pltpu.sync_copy(rows_v.at[0], acc.at[pl.ds(k * 128, 128)])

        def writeback(out_hbm):
            @pl.loop(0, 5)
            def _(j):
                k = sid + j * 16

                @pl.when(k < CPW)
                def _():
                    pltpu.sync_copy(acc.at[pl.ds(k * 128, 128)],
                                    out_hbm.at[cid].at[pl.ds(k * 128, 128)])

        def a_issue(c, buf):
            pltpu.async_copy(mi_hbm.at[pl.ds(c * 128, 128)], mi_v.at[buf], sin[buf])
            pltpu.async_copy(uj_hbm.at[pl.ds(c * 128, 128)], rows_v.at[buf], sin[buf])

        def a_wait(c, buf):
            pltpu.make_async_copy(mi_hbm.at[pl.ds(c * 128, 128)], mi_v.at[buf], sin[buf]).wait()
            pltpu.make_async_copy(uj_hbm.at[pl.ds(c * 128, 128)], rows_v.at[buf], sin[buf]).wait()

        fill0(0.0)
        zero_acc()
        plsc.subcore_barrier()
        a_issue(base, 0)

        @pl.loop(0, CPW)
        def _(j):
            c = base + j
            for buf in (0, 1):
                obuf = 1 - buf

                @pl.when(lax.rem(j, 2) == buf)
                def _():
                    a_wait(c, buf)

                    @pl.when(j + 1 < CPW)
                    def _():
                        a_issue(c + 1, obuf)

                    pltpu.sync_copy(rows_v.at[buf], acc.at[mi_v.at[buf]], add=True)

                    for o in range(8):
                        idx = mi_v[buf, pl.ds(o * 16, 16)]
                        ridx = lax.shift_right_logical(idx, 7)
                        lidx = lax.bitwise_and(idx, 127)
                        plsc.addupdate_scatter(cnt_v, [ridx, lidx],
                                               jnp.ones((16,), jnp.float32))

        pltpu.sync_copy(cnt_v, outc_hbm.at[wid])
        plsc.subcore_barrier()
        writeback(outf_hbm)

    return k4(uj, mi)


def _k5_body(pf0_ref, pf1_ref, cnt_ref, nf_ref, wcls_ref, bcls_ref, out_ref):
    sf = pf0_ref[...] + pf1_ref[...]
    cnt = cnt_ref[...]
    mean = sf / jnp.maximum(cnt, 1.0)
    agg = jnp.where(cnt > 0, mean, nf_ref[...])
    out_ref[...] = jnp.dot(agg, wcls_ref[...]) + bcls_ref[...]


def _k5(pf0, pf1, cnt, nf, W_cls, b_cls):
    grid = (N // NB,)
    return pl.pallas_call(
        _k5_body,
        grid=grid,
        in_specs=[
            pl.BlockSpec((NB, 128), lambda i: (i, 0)),
            pl.BlockSpec((NB, 128), lambda i: (i, 0)),
            pl.BlockSpec((NB, 1), lambda i: (i, 0)),
            pl.BlockSpec((NB, 128), lambda i: (i, 0)),
            pl.BlockSpec((128, 2), lambda i: (0, 0)),
            pl.BlockSpec((1, 2), lambda i: (0, 0)),
        ],
        out_specs=pl.BlockSpec((NB, 2), lambda i: (i, 0)),
        out_shape=jax.ShapeDtypeStruct((N, 2), jnp.float32),
    )(pf0, pf1, cnt, nf, W_cls, b_cls.reshape(1, 2))


def kernel(image_tensor, tweets_tensor, num_prop, category_prop, edge_index, edge_type,
           W_image, b_image, W_tweet, b_tweet, W_num, b_num, W_cat, b_cat,
           W_fusion, b_fusion, edge_type_embed, W_mu, b_mu, W_logvar, b_logvar,
           W_ec1, b_ec1, W_ec2, b_ec2, W_cls, b_cls):
    nf = _k1(image_tensor, tweets_tensor, num_prop, category_prop,
             W_image, b_image.reshape(1, H), W_tweet, b_tweet.reshape(1, H),
             W_num, b_num.reshape(1, H), W_cat, b_cat.reshape(1, H),
             W_fusion, b_fusion.reshape(1, H))

    src = edge_index[0]
    dst = edge_index[1]
    pad = EP - E
    src_p = jnp.pad(src, (0, pad))
    dst_p = jnp.pad(dst, (0, pad))
    et_p = jnp.pad(edge_type, (0, pad))

    g1, g2 = _k2(nf, src_p, dst_p)

    conf = _k3(g1, g2, et_p.reshape(EP, 1), edge_type_embed, W_ec1, b_ec1, W_ec2, b_ec2)

    _, k_bern = jax.random.split(jax.random.key(42))
    u = jax.random.uniform(k_bern, (E,), dtype=jnp.float32)
    u_p = jnp.pad(u, (0, pad), constant_values=2.0)
    p_ij = jax.nn.sigmoid(conf[:, 0])
    s = u_p < p_ij
    mi = jnp.where(s, src_p, DUMMY).astype(jnp.int32)

    pf, pc = _k4(g2, mi)

    cnt = jnp.sum(pc, axis=0).reshape(80 * 128)[:N].reshape(N, 1)

    return _k5(pf[0], pf[1], cnt, nf, W_cls, b_cls)
